# baseline (speedup 1.0000x reference)
import functools

import jax
import jax.numpy as jnp
from jax import lax
from jax.experimental import pallas as pl
from jax.experimental.pallas import tpu as pltpu
from jax.experimental.pallas import tpu_sc as plsc

N = 10000
E = 320000
D = 128
H = 256
T = 0.2
ALPHA = 0.5

NC, NS = 2, 16
NW = NC * NS
EPT = E // NW
KW = 125
NWIN = EPT // KW
HW = NWIN // 2
MN = 5000
MP = 5120
MPT = MP // NW
MKW = 80
NH = 10240
HPT = NH // NS
NA = 10240
APT = NA // NS
FL = 128

_MESH = dict(core_axis_name="c", subcore_axis_name="s")


def _wid():
    return lax.axis_index("s") * NC + lax.axis_index("c")


def _hist_body(src3, dst3, msk3, mupd3, ones_h, z_h,
               degs_o, degd_o, m01_o,
               sidx_v, didx_v, midx_v, mupd_v, ones_v, z_v, bounce_v,
               hs_sh, hd_sh, hm_sh):
    core = lax.axis_index("c")
    sid = lax.axis_index("s")
    wid = _wid()
    pltpu.sync_copy(z_h, z_v)
    pltpu.sync_copy(z_v, hs_sh.at[pl.ds(sid * HPT, HPT)])
    pltpu.sync_copy(z_v, hd_sh.at[pl.ds(sid * HPT, HPT)])
    pltpu.sync_copy(z_v, hm_sh.at[pl.ds(sid * HPT, HPT)])
    pltpu.sync_copy(ones_h, ones_v)
    pltpu.sync_copy(src3.at[wid], sidx_v)
    pltpu.sync_copy(dst3.at[wid], didx_v)
    pltpu.sync_copy(msk3.at[wid], midx_v)
    pltpu.sync_copy(mupd3.at[wid], mupd_v)
    plsc.subcore_barrier()

    def win(j, carry):
        pltpu.sync_copy(ones_v, hs_sh.at[sidx_v.at[j]], add=True)
        pltpu.sync_copy(ones_v, hd_sh.at[didx_v.at[j]], add=True)
        return carry

    lax.fori_loop(0, NWIN, win, 0)
    pltpu.sync_copy(mupd_v.at[0], hm_sh.at[midx_v.at[0]], add=True)
    pltpu.sync_copy(mupd_v.at[1], hm_sh.at[midx_v.at[1]], add=True)
    plsc.subcore_barrier()

    @pl.when(sid == 0)
    def _f0():
        pltpu.sync_copy(hs_sh, bounce_v)
        pltpu.sync_copy(bounce_v, degs_o.at[core])

    @pl.when(sid == 1)
    def _f1():
        pltpu.sync_copy(hd_sh, bounce_v)
        pltpu.sync_copy(bounce_v, degd_o.at[core])

    @pl.when(sid == 2)
    def _f2():
        pltpu.sync_copy(hm_sh, bounce_v)
        pltpu.sync_copy(bounce_v, m01_o.at[core])


@functools.cache
def _hist_kernel():
    return pl.kernel(
        _hist_body,
        out_type=(
            jax.ShapeDtypeStruct((NC, NH), jnp.float32),
            jax.ShapeDtypeStruct((NC, NH), jnp.float32),
            jax.ShapeDtypeStruct((NC, NH), jnp.float32),
        ),
        mesh=plsc.VectorSubcoreMesh(**_MESH),
        scratch_types=(
            pltpu.VMEM((NWIN, KW), jnp.int32),
            pltpu.VMEM((NWIN, KW), jnp.int32),
            pltpu.VMEM((MPT // MKW, MKW), jnp.int32),
            pltpu.VMEM((MPT // MKW, MKW), jnp.float32),
            pltpu.VMEM((KW,), jnp.float32),
            pltpu.VMEM((HPT,), jnp.float32),
            pltpu.VMEM((NH,), jnp.float32),
            pltpu.VMEM_SHARED((NH,), jnp.float32),
            pltpu.VMEM_SHARED((NH,), jnp.float32),
            pltpu.VMEM_SHARED((NH,), jnp.float32),
        ),
    )


def _make_prop(nchunks, ngather):
    def body(*refs):
        ys = refs[:nchunks]
        src3, dst3, z_h = refs[nchunks:nchunks + 3]
        k = nchunks + 3
        gidx_h = None
        gts = ()
        if ngather:
            gidx_h = refs[k]
            gts = refs[k + 1:k + 1 + ngather]
            k += 1 + ngather
        outs = refs[k:k + nchunks]
        k += nchunks
        gouts = refs[k:k + ngather]
        k += ngather
        sidx_v, didx_v, wbuf0_v, wbuf1_v, sem0, sem1 = refs[k:k + 6]
        if ngather:
            gidx_v = refs[k + 6]
        acc_sh = refs[-1]

        core = lax.axis_index("c")
        sid = lax.axis_index("s")
        wid = _wid()
        b0 = wbuf0_v.at[pl.ds(0, KW)]
        b1 = wbuf1_v.at[pl.ds(0, KW)]
        for c in range(nchunks):
            pltpu.sync_copy(z_h, wbuf0_v)
            for r in range(APT // FL):
                pltpu.sync_copy(
                    wbuf0_v, acc_sh.at[pl.ds(sid * APT + r * FL, FL)])
            plsc.subcore_barrier()
            for half in range(NWIN // HW):
                pltpu.sync_copy(src3.at[wid].at[pl.ds(half * HW, HW)], sidx_v)
                pltpu.sync_copy(dst3.at[wid].at[pl.ds(half * HW, HW)], didx_v)
                pltpu.async_copy(ys[c].at[sidx_v.at[0]], b0, sem0)

                def pair(i, carry):
                    j0 = 2 * i
                    pltpu.async_copy(ys[c].at[sidx_v.at[j0 + 1]], b1, sem1)
                    pltpu.make_async_copy(
                        ys[c].at[sidx_v.at[j0]], b0, sem0).wait()
                    pltpu.sync_copy(b0, acc_sh.at[didx_v.at[j0]], add=True)

                    @pl.when(i < HW // 2 - 1)
                    def _nx():
                        pltpu.async_copy(
                            ys[c].at[sidx_v.at[j0 + 2]], b0, sem0)

                    pltpu.make_async_copy(
                        ys[c].at[sidx_v.at[j0 + 1]], b1, sem1).wait()
                    pltpu.sync_copy(b1, acc_sh.at[didx_v.at[j0 + 1]], add=True)
                    return carry

                lax.fori_loop(0, HW // 2, pair, 0)
            plsc.subcore_barrier()
            for r in range(APT // FL):
                rows = pl.ds(sid * APT + r * FL, FL)
                pltpu.sync_copy(acc_sh.at[rows], wbuf0_v)
                pltpu.sync_copy(wbuf0_v, outs[c].at[core].at[rows])
            plsc.subcore_barrier()
        if ngather:
            pltpu.sync_copy(gidx_h.at[pl.ds(wid * MPT, MPT)], gidx_v)
            g0 = wbuf0_v.at[pl.ds(0, MKW)]
            g1 = wbuf1_v.at[pl.ds(0, MKW)]
            gsrcs = [gts[t].at[gidx_v.at[pl.ds(j * MKW, MKW)]]
                     for t in range(ngather) for j in range(MPT // MKW)]
            gdsts = [gouts[t].at[pl.ds(wid * MPT + j * MKW, MKW)]
                     for t in range(ngather) for j in range(MPT // MKW)]
            bufs = [g0, g1]
            sems = [sem0, sem1]
            pltpu.async_copy(gsrcs[0], bufs[0], sems[0])
            for i in range(len(gsrcs)):
                if i + 1 < len(gsrcs):
                    pltpu.async_copy(
                        gsrcs[i + 1], bufs[(i + 1) % 2], sems[(i + 1) % 2])
                pltpu.make_async_copy(gsrcs[i], bufs[i % 2], sems[i % 2]).wait()
                pltpu.sync_copy(bufs[i % 2], gdsts[i])

    out_type = tuple(
        jax.ShapeDtypeStruct((NC, NA, D), jnp.float32) for _ in range(nchunks)
    ) + tuple(
        jax.ShapeDtypeStruct((MP, D), jnp.float32) for _ in range(ngather)
    )
    scratch = [
        pltpu.VMEM((HW, KW), jnp.int32),
        pltpu.VMEM((HW, KW), jnp.int32),
        pltpu.VMEM((FL, D), jnp.float32),
        pltpu.VMEM((FL, D), jnp.float32),
        pltpu.SemaphoreType.DMA,
        pltpu.SemaphoreType.DMA,
    ]
    if ngather:
        scratch.append(pltpu.VMEM((MPT,), jnp.int32))
    scratch.append(pltpu.VMEM_SHARED((NA, D), jnp.float32))
    return pl.kernel(
        body,
        out_type=out_type,
        mesh=plsc.VectorSubcoreMesh(**_MESH),
        scratch_types=tuple(scratch),
    )


_make_prop = functools.cache(_make_prop)


def _hist_call(*args):
    return _hist_kernel()(*args)


def _prop2(*args):
    return _make_prop(2, 0)(*args)


def _prop1g2(*args):
    return _make_prop(1, 2)(*args)


BR = 1000
NB = N // BR
BM = 512
NBM = MP // BM

_f32 = jnp.float32


def _row(shape):
    return pl.BlockSpec(shape, lambda p, i: (i, 0))


def _full2(shape):
    return pl.BlockSpec(shape, lambda p, i: tuple(0 for _ in shape))


def _pp(shape):
    return pl.BlockSpec(shape, lambda p, i: (0, i, 0))


def _prelu_(x, a_ref):
    return jnp.where(x >= 0, x, a_ref[0, 0] * x)


def _k1_body(sa, sb, dja, djb, ma, mb, feat_b, mt,
             ns_o, nd_o, m01_o, y0_o, y1_o):
    ns = jnp.clip(sa[...] + sb[...], 1.0, None) ** -0.5
    nd = jnp.clip(dja[...] + djb[...], 1.0, None) ** -0.5
    m01 = ma[...] + mb[...]
    f = feat_b[...]
    x = f * (1.0 - m01) + m01 * mt[...]
    ns_o[...] = ns
    nd_o[...] = nd
    m01_o[...] = m01
    y0_o[...] = x * ns
    y1_o[...] = f * ns


def _k1_call(*arrs):
    return pl.pallas_call(
        _k1_body,
        grid=(1, NB),
        in_specs=[_row((BR, 1))] * 6 + [_row((BR, D)), _full2((1, D))],
        out_specs=[_row((BR, 1))] * 3 + [_row((BR, D))] * 2,
        out_shape=[jax.ShapeDtypeStruct((N, 1), _f32)] * 3
        + [jax.ShapeDtypeStruct((N, D), _f32)] * 2,
    )(*arrs)


def _bn_stats_acc(st, pre, r0):
    st[r0:r0 + 1, :] += jnp.sum(pre, 0, keepdims=True)
    st[r0 + 1:r0 + 2, :] += jnp.sum(pre * pre, 0, keepdims=True)


def _bn_apply(st, pre, r0, g, be, a):
    m = st[r0:r0 + 1, :] / N
    v = st[r0 + 1:r0 + 2, :] / N - m * m
    h = (pre - m) * lax.rsqrt(v + 1e-5) * g[...] + be[...]
    return _prelu_(h, a)


def _k2_body(p0, p1, ns, nd, W1, b1, g1, be1, a1, W2,
             tW1, tb1, tg1, tbe1, ta1, tW2, y2a_o, y2b_o, st):
    p = pl.program_id(0)
    i = pl.program_id(1)

    @pl.when((p == 0) & (i == 0))
    def _z():
        st[...] = jnp.zeros_like(st)

    ndb = nd[...]
    prex = (ndb * (p0[0] + p0[1])) @ W1[...] + b1[...]
    pref = (ndb * (p1[0] + p1[1])) @ tW1[...] + tb1[...]

    @pl.when(p == 0)
    def _acc():
        _bn_stats_acc(st, prex, 0)
        _bn_stats_acc(st, pref, 2)

    @pl.when(p == 1)
    def _apply():
        nsb = ns[...]
        e1 = _bn_apply(st, prex, 0, g1, be1, a1)
        te1 = _bn_apply(st, pref, 2, tg1, tbe1, ta1)
        y2a_o[...] = (e1 * nsb) @ W2[...]
        y2b_o[...] = (te1 * nsb) @ tW2[...]


def _k2_call(p0, p1, ns, nd, *ws):
    return pl.pallas_call(
        _k2_body,
        grid=(2, NB),
        in_specs=[_pp((NC, BR, D))] * 2 + [_row((BR, 1))] * 2
        + [_full2((D, H)), _full2((1, H)), _full2((1, H)), _full2((1, H)),
           _full2((1, 1)), _full2((H, D))] * 2,
        out_specs=[_row((BR, D))] * 2,
        out_shape=[jax.ShapeDtypeStruct((N, D), _f32)] * 2,
        scratch_shapes=[pltpu.VMEM((8, H), _f32)],
    )(p0, p1, ns, nd, *ws)


def _k3_body(q0, q1, ns, nd, m01, b2, g2, be2, a2, tb2, tg2, tbe2, ta2, dW,
             o_o, h2_o, y3_o, st):
    p = pl.program_id(0)
    i = pl.program_id(1)

    @pl.when((p == 0) & (i == 0))
    def _z():
        st[...] = jnp.zeros_like(st)

    ndb = nd[...]
    preo = ndb * (q0[0] + q0[1]) + b2[...]
    preh = ndb * (q1[0] + q1[1]) + tb2[...]

    @pl.when(p == 0)
    def _acc():
        _bn_stats_acc(st, preo, 0)
        _bn_stats_acc(st, preh, 2)

    @pl.when(p == 1)
    def _apply():
        o = _bn_apply(st, preo, 0, g2, be2, a2)
        h2 = _bn_apply(st, preh, 2, tg2, tbe2, ta2)
        o_o[...] = o
        h2_o[...] = h2
        y3_o[...] = ((o * (1.0 - m01[...])) * ns[...]) @ dW[...]


def _k3_call(q0, q1, ns, nd, m01, *ws):
    return pl.pallas_call(
        _k3_body,
        grid=(2, NB),
        in_specs=[_pp((NC, BR, D))] * 2 + [_row((BR, 1))] * 3
        + [_full2((1, D))] * 3 + [_full2((1, 1))]
        + [_full2((1, D))] * 3 + [_full2((1, 1))] + [_full2((D, D))],
        out_specs=[_row((BR, D))] * 3,
        out_shape=[jax.ShapeDtypeStruct((N, D), _f32)] * 3,
        scratch_shapes=[pltpu.VMEM((8, D), _f32)],
    )(q0, q1, ns, nd, m01, *ws)


def _k4_body(r0, nd, m01, feat_b, db, dg, dbe, da, loss_o, st, acc):
    p = pl.program_id(0)
    i = pl.program_id(1)

    @pl.when((p == 0) & (i == 0))
    def _z():
        st[...] = jnp.zeros_like(st)
        acc[0, 0] = 0.0

    u = nd[...] * (r0[0] + r0[1]) + db[...]

    @pl.when(p == 0)
    def _acc():
        _bn_stats_acc(st, u, 0)

    @pl.when(p == 1)
    def _apply():
        re = _bn_apply(st, u, 0, dg, dbe, da)
        fb = feat_b[...]
        rn = jnp.maximum(jnp.sqrt(jnp.sum(re * re, 1, keepdims=True)), 1e-12)
        fn = jnp.maximum(jnp.sqrt(jnp.sum(fb * fb, 1, keepdims=True)), 1e-12)
        cos = jnp.sum(re * fb, 1, keepdims=True) / (rn * fn)
        acc[0, 0] += jnp.sum(m01[...] * (1.0 - cos))

    @pl.when((p == 1) & (i == NB - 1))
    def _fin():
        loss_o[...] = jnp.full((1, 1), acc[0, 0] / MN, _f32)


def _k4_call(r0, nd, m01, feat, db, dg, dbe, da):
    return pl.pallas_call(
        _k4_body,
        grid=(2, NB),
        in_specs=[_pp((NC, BR, D))] + [_row((BR, 1))] * 2 + [_row((BR, D))]
        + [_full2((1, D))] * 3 + [_full2((1, 1))],
        out_specs=pl.BlockSpec((1, 1), lambda p, i: (0, 0)),
        out_shape=jax.ShapeDtypeStruct((1, 1), _f32),
        scratch_shapes=[pltpu.VMEM((8, D), _f32),
                        pltpu.SMEM((1, 1), _f32)],
    )(r0, nd, m01, feat, db, dg, dbe, da)


def _head(xb, W1_, b1_, W2_, b2_):
    t = jnp.maximum(xb @ W1_[...] + b1_[...], 0.0)
    c = t @ W2_[...] + b2_[...]
    n = jnp.maximum(jnp.sqrt(jnp.sum(c * c, 1, keepdims=True)), 1e-12)
    return c / n


def _k5_body(hm, om, pW1, pb1, pW2, pb2, qW1, qb1, qW2, qb2, nh_o, nm_o):
    nh_o[...] = _head(hm[...], pW1, pb1, pW2, pb2)
    nm_o[...] = _head(om[...], qW1, qb1, qW2, qb2)


def _k5_call(hm, om, *ws):
    return pl.pallas_call(
        _k5_body,
        grid=(1, NBM),
        in_specs=[_row((BM, D))] * 2
        + [_full2((D, H)), _full2((1, H)), _full2((H, D)), _full2((1, D))] * 2,
        out_specs=[_row((BM, D))] * 2,
        out_shape=[jax.ShapeDtypeStruct((MP, D), _f32)] * 2,
    )(hm, om, *ws)


def _k6_body(nh_b, nm_full, nm_b, loss1, out_o, acc):
    i = pl.program_id(1)

    @pl.when(i == 0)
    def _z():
        acc[0, 0] = 0.0

    a = nh_b[...]
    s = lax.dot_general(a, nm_full[...], (((1,), (1,)), ((), ())),
                        preferred_element_type=_f32) / T
    sim = jnp.exp(s)
    colm = (lax.broadcasted_iota(jnp.int32, (BM, MP), 1) < MN).astype(_f32)
    rowsum = jnp.sum(sim * colm, 1, keepdims=True)
    pos = jnp.exp(jnp.sum(a * nm_b[...], 1, keepdims=True) / T)
    rowid = i * BM + lax.broadcasted_iota(jnp.int32, (BM, 1), 0)
    valid = (rowid < MN).astype(_f32)
    den = jnp.where(valid > 0, rowsum - pos, 1.0)
    acc[0, 0] += jnp.sum(valid * (jnp.log(den) - jnp.log(pos)))

    @pl.when(i == NBM - 1)
    def _fin():
        out_o[...] = loss1[...] + ALPHA * jnp.full((1, 1), acc[0, 0] / MN, _f32)


def _k6_call(nh, nm, loss1):
    return pl.pallas_call(
        _k6_body,
        grid=(1, NBM),
        in_specs=[_row((BM, D)), _full2((MP, D)), _row((BM, D)),
                  _full2((1, 1))],
        out_specs=pl.BlockSpec((1, 1), lambda p, i: (0, 0)),
        out_shape=jax.ShapeDtypeStruct((1, 1), _f32),
        scratch_shapes=[pltpu.SMEM((1, 1), _f32)],
    )(nh, nm, nm, loss1)


def kernel(feat, edge_index, mask_nodes, W1, b1, g1, be1, a1, W2, b2, g2, be2,
           a2, tW1, tb1, tg1, tbe1, ta1, tW2, tb2, tg2, tbe2, ta2,
           dW, db, dg, dbe, da, mask_token,
           pW1, pb1, pW2, pb2, qW1, qb1, qW2, qb2):
    src3 = edge_index[0].reshape(NW, NWIN, KW)
    dst3 = edge_index[1].reshape(NW, NWIN, KW)
    mpad = jnp.concatenate(
        [mask_nodes, jnp.zeros((MP - MN,), jnp.int32)])
    msk3 = mpad.reshape(NW, MPT // MKW, MKW)
    mupd3 = jnp.concatenate(
        [jnp.ones((MN,), jnp.float32), jnp.zeros((MP - MN,), jnp.float32)]
    ).reshape(NW, MPT // MKW, MKW)
    ones_h = jnp.ones((KW,), jnp.float32)
    zh_h = jnp.zeros((HPT,), jnp.float32)
    zr_h = jnp.zeros((FL, D), jnp.float32)

    degs2, degd2, m012 = _hist_call(src3, dst3, msk3, mupd3, ones_h, zh_h)
    col = lambda a: a.reshape(NH, 1)[:N]
    row = lambda a: a.reshape(1, -1)
    sc = lambda a: a.reshape(1, 1)

    ns, nd, m01, y0, y1 = _k1_call(
        col(degs2[0]), col(degs2[1]), col(degd2[0]), col(degd2[1]),
        col(m012[0]), col(m012[1]), feat, mask_token)

    p0, p1 = _prop2(y0, y1, src3, dst3, zr_h)
    y2a, y2b = _k2_call(
        p0, p1, ns, nd,
        W1, row(b1), row(g1), row(be1), sc(a1), W2,
        tW1, row(tb1), row(tg1), row(tbe1), sc(ta1), tW2)

    q0, q1 = _prop2(y2a, y2b, src3, dst3, zr_h)
    o, h2, y3 = _k3_call(
        q0, q1, ns, nd, m01,
        row(b2), row(g2), row(be2), sc(a2),
        row(tb2), row(tg2), row(tbe2), sc(ta2), dW)

    r0, om, hm = _prop1g2(y3, src3, dst3, zr_h, mpad, o, h2)
    loss1 = _k4_call(r0, nd, m01, feat, row(db), row(dg), row(dbe), sc(da))

    nh, nm = _k5_call(hm, om, pW1, row(pb1), pW2, row(pb2),
                      qW1, row(qb1), qW2, row(qb2))
    total = _k6_call(nh, nm, loss1)
    return total[0, 0]


# speedup vs baseline: 1.0031x; 1.0031x over previous
import functools

import jax
import jax.numpy as jnp
from jax import lax
from jax.experimental import pallas as pl
from jax.experimental.pallas import tpu as pltpu
from jax.experimental.pallas import tpu_sc as plsc

N = 10000
E = 320000
D = 128
H = 256
T = 0.2
ALPHA = 0.5

NC, NS = 2, 16
NW = NC * NS
EPT = E // NW
KW = 128
NWIN = 80
EPP = NWIN * KW
HW = NWIN // 2
MN = 5000
MP = 5120
MPT = MP // NW
MKW = 80
NH = 10240
HPT = NH // NS
NA = 10240
APT = NA // NS
FL = 128

_MESH = dict(core_axis_name="c", subcore_axis_name="s")


def _wid():
    return lax.axis_index("s") * NC + lax.axis_index("c")


def _hist_body(src3, dst3, msk3, mupd3, ones_h, z_h,
               degs_o, degd_o, m01_o,
               sidx_v, didx_v, midx_v, mupd_v, ones_v, z_v, bounce_v,
               hs_sh, hd_sh, hm_sh):
    core = lax.axis_index("c")
    sid = lax.axis_index("s")
    wid = _wid()
    pltpu.sync_copy(z_h, z_v)
    pltpu.sync_copy(z_v, hs_sh.at[pl.ds(sid * HPT, HPT)])
    pltpu.sync_copy(z_v, hd_sh.at[pl.ds(sid * HPT, HPT)])
    pltpu.sync_copy(z_v, hm_sh.at[pl.ds(sid * HPT, HPT)])
    pltpu.sync_copy(ones_h, ones_v)
    pltpu.sync_copy(src3.at[wid], sidx_v)
    pltpu.sync_copy(dst3.at[wid], didx_v)
    pltpu.sync_copy(msk3.at[wid], midx_v)
    pltpu.sync_copy(mupd3.at[wid], mupd_v)
    plsc.subcore_barrier()

    def win(j, carry):
        pltpu.sync_copy(ones_v, hs_sh.at[sidx_v.at[j]], add=True)
        pltpu.sync_copy(ones_v, hd_sh.at[didx_v.at[j]], add=True)
        return carry

    lax.fori_loop(0, NWIN, win, 0)
    pltpu.sync_copy(mupd_v.at[0], hm_sh.at[midx_v.at[0]], add=True)
    pltpu.sync_copy(mupd_v.at[1], hm_sh.at[midx_v.at[1]], add=True)
    plsc.subcore_barrier()

    @pl.when(sid == 0)
    def _f0():
        pltpu.sync_copy(hs_sh, bounce_v)
        pltpu.sync_copy(bounce_v, degs_o.at[core])

    @pl.when(sid == 1)
    def _f1():
        pltpu.sync_copy(hd_sh, bounce_v)
        pltpu.sync_copy(bounce_v, degd_o.at[core])

    @pl.when(sid == 2)
    def _f2():
        pltpu.sync_copy(hm_sh, bounce_v)
        pltpu.sync_copy(bounce_v, m01_o.at[core])


@functools.cache
def _hist_kernel():
    return pl.kernel(
        _hist_body,
        out_type=(
            jax.ShapeDtypeStruct((NC, NH), jnp.float32),
            jax.ShapeDtypeStruct((NC, NH), jnp.float32),
            jax.ShapeDtypeStruct((NC, NH), jnp.float32),
        ),
        mesh=plsc.VectorSubcoreMesh(**_MESH),
        scratch_types=(
            pltpu.VMEM((NWIN, KW), jnp.int32),
            pltpu.VMEM((NWIN, KW), jnp.int32),
            pltpu.VMEM((MPT // MKW, MKW), jnp.int32),
            pltpu.VMEM((MPT // MKW, MKW), jnp.float32),
            pltpu.VMEM((KW,), jnp.float32),
            pltpu.VMEM((HPT,), jnp.float32),
            pltpu.VMEM((NH,), jnp.float32),
            pltpu.VMEM_SHARED((NH,), jnp.float32),
            pltpu.VMEM_SHARED((NH,), jnp.float32),
            pltpu.VMEM_SHARED((NH,), jnp.float32),
        ),
    )


def _make_prop(nchunks, ngather):
    def body(*refs):
        ys = refs[:nchunks]
        src3, dst3, z_h = refs[nchunks:nchunks + 3]
        k = nchunks + 3
        gidx_h = None
        gts = ()
        if ngather:
            gidx_h = refs[k]
            gts = refs[k + 1:k + 1 + ngather]
            k += 1 + ngather
        outs = refs[k:k + nchunks]
        k += nchunks
        gouts = refs[k:k + ngather]
        k += ngather
        sidx_v, didx_v, wbuf0_v, wbuf1_v, sem0, sem1 = refs[k:k + 6]
        if ngather:
            gidx_v = refs[k + 6]
        acc_sh = refs[-1]

        core = lax.axis_index("c")
        sid = lax.axis_index("s")
        wid = _wid()
        b0 = wbuf0_v
        b1 = wbuf1_v
        for c in range(nchunks):
            pltpu.sync_copy(z_h, wbuf0_v)
            for r in range(APT // FL):
                pltpu.sync_copy(
                    wbuf0_v, acc_sh.at[pl.ds(sid * APT + r * FL, FL)])
            plsc.subcore_barrier()
            for half in range(NWIN // HW):
                pltpu.sync_copy(src3.at[wid].at[pl.ds(half * HW, HW)], sidx_v)
                pltpu.sync_copy(dst3.at[wid].at[pl.ds(half * HW, HW)], didx_v)
                pltpu.async_copy(ys[c].at[sidx_v.at[0]], b0, sem0)

                def pair(i, carry):
                    j0 = 2 * i
                    pltpu.async_copy(ys[c].at[sidx_v.at[j0 + 1]], b1, sem1)
                    pltpu.make_async_copy(
                        ys[c].at[sidx_v.at[j0]], b0, sem0).wait()
                    pltpu.sync_copy(b0, acc_sh.at[didx_v.at[j0]], add=True)

                    @pl.when(i < HW // 2 - 1)
                    def _nx():
                        pltpu.async_copy(
                            ys[c].at[sidx_v.at[j0 + 2]], b0, sem0)

                    pltpu.make_async_copy(
                        ys[c].at[sidx_v.at[j0 + 1]], b1, sem1).wait()
                    pltpu.sync_copy(b1, acc_sh.at[didx_v.at[j0 + 1]], add=True)
                    return carry

                lax.fori_loop(0, HW // 2, pair, 0)
            plsc.subcore_barrier()
            for r in range(APT // FL):
                rows = pl.ds(sid * APT + r * FL, FL)
                pltpu.sync_copy(acc_sh.at[rows], wbuf0_v)
                pltpu.sync_copy(wbuf0_v, outs[c].at[core].at[rows])
            plsc.subcore_barrier()
        if ngather:
            pltpu.sync_copy(gidx_h.at[pl.ds(wid * MPT, MPT)], gidx_v)
            g0 = wbuf0_v.at[pl.ds(0, MKW)]
            g1 = wbuf1_v.at[pl.ds(0, MKW)]
            gsrcs = [gts[t].at[gidx_v.at[pl.ds(j * MKW, MKW)]]
                     for t in range(ngather) for j in range(MPT // MKW)]
            gdsts = [gouts[t].at[pl.ds(wid * MPT + j * MKW, MKW)]
                     for t in range(ngather) for j in range(MPT // MKW)]
            bufs = [g0, g1]
            sems = [sem0, sem1]
            pltpu.async_copy(gsrcs[0], bufs[0], sems[0])
            for i in range(len(gsrcs)):
                if i + 1 < len(gsrcs):
                    pltpu.async_copy(
                        gsrcs[i + 1], bufs[(i + 1) % 2], sems[(i + 1) % 2])
                pltpu.make_async_copy(gsrcs[i], bufs[i % 2], sems[i % 2]).wait()
                pltpu.sync_copy(bufs[i % 2], gdsts[i])

    out_type = tuple(
        jax.ShapeDtypeStruct((NC, NA, D), jnp.float32) for _ in range(nchunks)
    ) + tuple(
        jax.ShapeDtypeStruct((MP, D), jnp.float32) for _ in range(ngather)
    )
    scratch = [
        pltpu.VMEM((HW, KW), jnp.int32),
        pltpu.VMEM((HW, KW), jnp.int32),
        pltpu.VMEM((FL, D), jnp.float32),
        pltpu.VMEM((FL, D), jnp.float32),
        pltpu.SemaphoreType.DMA,
        pltpu.SemaphoreType.DMA,
    ]
    if ngather:
        scratch.append(pltpu.VMEM((MPT,), jnp.int32))
    scratch.append(pltpu.VMEM_SHARED((NA, D), jnp.float32))
    return pl.kernel(
        body,
        out_type=out_type,
        mesh=plsc.VectorSubcoreMesh(**_MESH),
        scratch_types=tuple(scratch),
    )


_make_prop = functools.cache(_make_prop)


def _hist_call(*args):
    return _hist_kernel()(*args)


def _prop2(*args):
    return _make_prop(2, 0)(*args)


def _prop1g2(*args):
    return _make_prop(1, 2)(*args)


BR = 2000
NB = N // BR
BM = 512
NBM = MP // BM

_f32 = jnp.float32


def _row(shape):
    return pl.BlockSpec(shape, lambda p, i: (i, 0))


def _full2(shape):
    return pl.BlockSpec(shape, lambda p, i: tuple(0 for _ in shape))


def _pp(shape):
    return pl.BlockSpec(shape, lambda p, i: (0, i, 0))


def _prelu_(x, a_ref):
    return jnp.where(x >= 0, x, a_ref[0, 0] * x)


def _k1_body(sa, sb, dja, djb, ma, mb, feat_b, mt,
             ns_o, nd_o, m01_o, y0_o, y1_o):
    ns = jnp.clip(sa[...] + sb[...], 1.0, None) ** -0.5
    nd = jnp.clip(dja[...] + djb[...], 1.0, None) ** -0.5
    m01 = ma[...] + mb[...]
    f = feat_b[...]
    x = f * (1.0 - m01) + m01 * mt[...]
    ns_o[...] = ns
    nd_o[...] = nd
    m01_o[...] = m01
    y0_o[...] = x * ns
    y1_o[...] = f * ns


def _k1_call(*arrs):
    return pl.pallas_call(
        _k1_body,
        grid=(1, NB),
        in_specs=[_row((BR, 1))] * 6 + [_row((BR, D)), _full2((1, D))],
        out_specs=[_row((BR, 1))] * 3 + [_row((BR, D))] * 2,
        out_shape=[jax.ShapeDtypeStruct((N, 1), _f32)] * 3
        + [jax.ShapeDtypeStruct((NA, D), _f32)] * 2,
    )(*arrs)


def _bn_stats_acc(st, pre, r0):
    st[r0:r0 + 1, :] += jnp.sum(pre, 0, keepdims=True)
    st[r0 + 1:r0 + 2, :] += jnp.sum(pre * pre, 0, keepdims=True)


def _bn_apply(st, pre, r0, g, be, a):
    m = st[r0:r0 + 1, :] / N
    v = st[r0 + 1:r0 + 2, :] / N - m * m
    h = (pre - m) * lax.rsqrt(v + 1e-5) * g[...] + be[...]
    return _prelu_(h, a)


def _k2_body(p0, p1, ns, nd, W1, b1, g1, be1, a1, W2,
             tW1, tb1, tg1, tbe1, ta1, tW2, y2a_o, y2b_o, st):
    p = pl.program_id(0)
    i = pl.program_id(1)

    @pl.when((p == 0) & (i == 0))
    def _z():
        st[...] = jnp.zeros_like(st)

    ndb = nd[...]
    prex = (ndb * (p0[0] + p0[1])) @ W1[...] + b1[...]
    pref = (ndb * (p1[0] + p1[1])) @ tW1[...] + tb1[...]

    @pl.when(p == 0)
    def _acc():
        _bn_stats_acc(st, prex, 0)
        _bn_stats_acc(st, pref, 2)

    @pl.when(p == 1)
    def _apply():
        nsb = ns[...]
        e1 = _bn_apply(st, prex, 0, g1, be1, a1)
        te1 = _bn_apply(st, pref, 2, tg1, tbe1, ta1)
        y2a_o[...] = (e1 * nsb) @ W2[...]
        y2b_o[...] = (te1 * nsb) @ tW2[...]


def _k2_call(p0, p1, ns, nd, *ws):
    return pl.pallas_call(
        _k2_body,
        grid=(2, NB),
        in_specs=[_pp((NC, BR, D))] * 2 + [_row((BR, 1))] * 2
        + [_full2((D, H)), _full2((1, H)), _full2((1, H)), _full2((1, H)),
           _full2((1, 1)), _full2((H, D))] * 2,
        out_specs=[_row((BR, D))] * 2,
        out_shape=[jax.ShapeDtypeStruct((NA, D), _f32)] * 2,
        scratch_shapes=[pltpu.VMEM((8, H), _f32)],
    )(p0, p1, ns, nd, *ws)


def _k3_body(q0, q1, ns, nd, m01, b2, g2, be2, a2, tb2, tg2, tbe2, ta2, dW,
             o_o, h2_o, y3_o, st):
    p = pl.program_id(0)
    i = pl.program_id(1)

    @pl.when((p == 0) & (i == 0))
    def _z():
        st[...] = jnp.zeros_like(st)

    ndb = nd[...]
    preo = ndb * (q0[0] + q0[1]) + b2[...]
    preh = ndb * (q1[0] + q1[1]) + tb2[...]

    @pl.when(p == 0)
    def _acc():
        _bn_stats_acc(st, preo, 0)
        _bn_stats_acc(st, preh, 2)

    @pl.when(p == 1)
    def _apply():
        o = _bn_apply(st, preo, 0, g2, be2, a2)
        h2 = _bn_apply(st, preh, 2, tg2, tbe2, ta2)
        o_o[...] = o
        h2_o[...] = h2
        y3_o[...] = ((o * (1.0 - m01[...])) * ns[...]) @ dW[...]


def _k3_call(q0, q1, ns, nd, m01, *ws):
    return pl.pallas_call(
        _k3_body,
        grid=(2, NB),
        in_specs=[_pp((NC, BR, D))] * 2 + [_row((BR, 1))] * 3
        + [_full2((1, D))] * 3 + [_full2((1, 1))]
        + [_full2((1, D))] * 3 + [_full2((1, 1))] + [_full2((D, D))],
        out_specs=[_row((BR, D))] * 3,
        out_shape=[jax.ShapeDtypeStruct((NA, D), _f32)] * 3,
        scratch_shapes=[pltpu.VMEM((8, D), _f32)],
    )(q0, q1, ns, nd, m01, *ws)


def _k4_body(r0, nd, m01, feat_b, db, dg, dbe, da, loss_o, st, acc):
    p = pl.program_id(0)
    i = pl.program_id(1)

    @pl.when((p == 0) & (i == 0))
    def _z():
        st[...] = jnp.zeros_like(st)
        acc[0, 0] = 0.0

    u = nd[...] * (r0[0] + r0[1]) + db[...]

    @pl.when(p == 0)
    def _acc():
        _bn_stats_acc(st, u, 0)

    @pl.when(p == 1)
    def _apply():
        re = _bn_apply(st, u, 0, dg, dbe, da)
        fb = feat_b[...]
        rn = jnp.maximum(jnp.sqrt(jnp.sum(re * re, 1, keepdims=True)), 1e-12)
        fn = jnp.maximum(jnp.sqrt(jnp.sum(fb * fb, 1, keepdims=True)), 1e-12)
        cos = jnp.sum(re * fb, 1, keepdims=True) / (rn * fn)
        acc[0, 0] += jnp.sum(m01[...] * (1.0 - cos))

    @pl.when((p == 1) & (i == NB - 1))
    def _fin():
        loss_o[...] = jnp.full((1, 1), acc[0, 0] / MN, _f32)


def _k4_call(r0, nd, m01, feat, db, dg, dbe, da):
    return pl.pallas_call(
        _k4_body,
        grid=(2, NB),
        in_specs=[_pp((NC, BR, D))] + [_row((BR, 1))] * 2 + [_row((BR, D))]
        + [_full2((1, D))] * 3 + [_full2((1, 1))],
        out_specs=pl.BlockSpec((1, 1), lambda p, i: (0, 0)),
        out_shape=jax.ShapeDtypeStruct((1, 1), _f32),
        scratch_shapes=[pltpu.VMEM((8, D), _f32),
                        pltpu.SMEM((1, 1), _f32)],
    )(r0, nd, m01, feat, db, dg, dbe, da)


def _head(xb, W1_, b1_, W2_, b2_):
    t = jnp.maximum(xb @ W1_[...] + b1_[...], 0.0)
    c = t @ W2_[...] + b2_[...]
    n = jnp.maximum(jnp.sqrt(jnp.sum(c * c, 1, keepdims=True)), 1e-12)
    return c / n


def _k5_body(hm, om, pW1, pb1, pW2, pb2, qW1, qb1, qW2, qb2, nh_o, nm_o):
    nh_o[...] = _head(hm[...], pW1, pb1, pW2, pb2)
    nm_o[...] = _head(om[...], qW1, qb1, qW2, qb2)


def _k5_call(hm, om, *ws):
    return pl.pallas_call(
        _k5_body,
        grid=(1, NBM),
        in_specs=[_row((BM, D))] * 2
        + [_full2((D, H)), _full2((1, H)), _full2((H, D)), _full2((1, D))] * 2,
        out_specs=[_row((BM, D))] * 2,
        out_shape=[jax.ShapeDtypeStruct((MP, D), _f32)] * 2,
    )(hm, om, *ws)


def _k6_body(nh_b, nm_full, nm_b, loss1, out_o, acc):
    i = pl.program_id(1)

    @pl.when(i == 0)
    def _z():
        acc[0, 0] = 0.0

    a = nh_b[...]
    s = lax.dot_general(a, nm_full[...], (((1,), (1,)), ((), ())),
                        preferred_element_type=_f32) / T
    sim = jnp.exp(s)
    colm = (lax.broadcasted_iota(jnp.int32, (BM, MP), 1) < MN).astype(_f32)
    rowsum = jnp.sum(sim * colm, 1, keepdims=True)
    pos = jnp.exp(jnp.sum(a * nm_b[...], 1, keepdims=True) / T)
    rowid = i * BM + lax.broadcasted_iota(jnp.int32, (BM, 1), 0)
    valid = (rowid < MN).astype(_f32)
    den = jnp.where(valid > 0, rowsum - pos, 1.0)
    acc[0, 0] += jnp.sum(valid * (jnp.log(den) - jnp.log(pos)))

    @pl.when(i == NBM - 1)
    def _fin():
        out_o[...] = loss1[...] + ALPHA * jnp.full((1, 1), acc[0, 0] / MN, _f32)


def _k6_call(nh, nm, loss1):
    return pl.pallas_call(
        _k6_body,
        grid=(1, NBM),
        in_specs=[_row((BM, D)), _full2((MP, D)), _row((BM, D)),
                  _full2((1, 1))],
        out_specs=pl.BlockSpec((1, 1), lambda p, i: (0, 0)),
        out_shape=jax.ShapeDtypeStruct((1, 1), _f32),
        scratch_shapes=[pltpu.SMEM((1, 1), _f32)],
    )(nh, nm, nm, loss1)


def kernel(feat, edge_index, mask_nodes, W1, b1, g1, be1, a1, W2, b2, g2, be2,
           a2, tW1, tb1, tg1, tbe1, ta1, tW2, tb2, tg2, tbe2, ta2,
           dW, db, dg, dbe, da, mask_token,
           pW1, pb1, pW2, pb2, qW1, qb1, qW2, qb2):
    dummy = (N + jnp.arange(EPP - EPT, dtype=jnp.int32))[None, :]
    dummy = jnp.broadcast_to(dummy, (NW, EPP - EPT))
    src3 = jnp.concatenate(
        [edge_index[0].reshape(NW, EPT), dummy], axis=1).reshape(NW, NWIN, KW)
    dst3 = jnp.concatenate(
        [edge_index[1].reshape(NW, EPT), dummy], axis=1).reshape(NW, NWIN, KW)
    mpad = jnp.concatenate(
        [mask_nodes, jnp.zeros((MP - MN,), jnp.int32)])
    msk3 = mpad.reshape(NW, MPT // MKW, MKW)
    mupd3 = jnp.concatenate(
        [jnp.ones((MN,), jnp.float32), jnp.zeros((MP - MN,), jnp.float32)]
    ).reshape(NW, MPT // MKW, MKW)
    ones_h = jnp.ones((KW,), jnp.float32)
    zh_h = jnp.zeros((HPT,), jnp.float32)
    zr_h = jnp.zeros((FL, D), jnp.float32)

    degs2, degd2, m012 = _hist_call(src3, dst3, msk3, mupd3, ones_h, zh_h)
    col = lambda a: a.reshape(NH, 1)[:N]
    row = lambda a: a.reshape(1, -1)
    sc = lambda a: a.reshape(1, 1)

    ns, nd, m01, y0, y1 = _k1_call(
        col(degs2[0]), col(degs2[1]), col(degd2[0]), col(degd2[1]),
        col(m012[0]), col(m012[1]), feat, mask_token)

    p0, p1 = _prop2(y0, y1, src3, dst3, zr_h)
    y2a, y2b = _k2_call(
        p0, p1, ns, nd,
        W1, row(b1), row(g1), row(be1), sc(a1), W2,
        tW1, row(tb1), row(tg1), row(tbe1), sc(ta1), tW2)

    q0, q1 = _prop2(y2a, y2b, src3, dst3, zr_h)
    o, h2, y3 = _k3_call(
        q0, q1, ns, nd, m01,
        row(b2), row(g2), row(be2), sc(a2),
        row(tb2), row(tg2), row(tbe2), sc(ta2), dW)

    r0, om, hm = _prop1g2(y3, src3, dst3, zr_h, mpad, o, h2)
    loss1 = _k4_call(r0, nd, m01, feat, row(db), row(dg), row(dbe), sc(da))

    nh, nm = _k5_call(hm, om, pW1, row(pb1), pW2, row(pb2),
                      qW1, row(qb1), qW2, row(qb2))
    total = _k6_call(nh, nm, loss1)
    return total[0, 0]


# speedup vs baseline: 1.0555x; 1.0523x over previous
import functools

import jax
import jax.numpy as jnp
from jax import lax
from jax.experimental import pallas as pl
from jax.experimental.pallas import tpu as pltpu
from jax.experimental.pallas import tpu_sc as plsc

N = 10000
E = 320000
D = 128
H = 256
T = 0.2
ALPHA = 0.5

NC, NS = 2, 16
NW = NC * NS
EPT = E // NW
KW = 128
NWIN = 80
EPP = NWIN * KW
HW = NWIN // 2
MN = 5000
MP = 5120
MPT = MP // NW
MKW = 80
NH = 10240
HPT = NH // NS
NA = 10240
APT = NA // NS
FL = 128

_MESH = dict(core_axis_name="c", subcore_axis_name="s")


def _wid():
    return lax.axis_index("s") * NC + lax.axis_index("c")


def _hist_body(src3, dst3, msk3, mupd3, ones_h, z_h,
               degs_o, degd_o, m01_o,
               sidx_v, didx_v, midx_v, mupd_v, ones_v, z_v, bounce_v,
               hs_sh, hd_sh, hm_sh):
    core = lax.axis_index("c")
    sid = lax.axis_index("s")
    wid = _wid()
    pltpu.sync_copy(z_h, z_v)
    pltpu.sync_copy(z_v, hs_sh.at[pl.ds(sid * HPT, HPT)])
    pltpu.sync_copy(z_v, hd_sh.at[pl.ds(sid * HPT, HPT)])
    pltpu.sync_copy(z_v, hm_sh.at[pl.ds(sid * HPT, HPT)])
    pltpu.sync_copy(ones_h, ones_v)
    pltpu.sync_copy(src3.at[wid], sidx_v)
    pltpu.sync_copy(dst3.at[wid], didx_v)
    pltpu.sync_copy(msk3.at[wid], midx_v)
    pltpu.sync_copy(mupd3.at[wid], mupd_v)
    plsc.subcore_barrier()

    def win(j, carry):
        pltpu.sync_copy(ones_v, hs_sh.at[sidx_v.at[j]], add=True)
        pltpu.sync_copy(ones_v, hd_sh.at[didx_v.at[j]], add=True)
        return carry

    lax.fori_loop(0, NWIN, win, 0)
    pltpu.sync_copy(mupd_v.at[0], hm_sh.at[midx_v.at[0]], add=True)
    pltpu.sync_copy(mupd_v.at[1], hm_sh.at[midx_v.at[1]], add=True)
    plsc.subcore_barrier()

    @pl.when(sid == 0)
    def _f0():
        pltpu.sync_copy(hs_sh, bounce_v)
        pltpu.sync_copy(bounce_v, degs_o.at[core])

    @pl.when(sid == 1)
    def _f1():
        pltpu.sync_copy(hd_sh, bounce_v)
        pltpu.sync_copy(bounce_v, degd_o.at[core])

    @pl.when(sid == 2)
    def _f2():
        pltpu.sync_copy(hm_sh, bounce_v)
        pltpu.sync_copy(bounce_v, m01_o.at[core])


@functools.cache
def _hist_kernel():
    return pl.kernel(
        _hist_body,
        out_type=(
            jax.ShapeDtypeStruct((NC, NH), jnp.float32),
            jax.ShapeDtypeStruct((NC, NH), jnp.float32),
            jax.ShapeDtypeStruct((NC, NH), jnp.float32),
        ),
        mesh=plsc.VectorSubcoreMesh(**_MESH),
        scratch_types=(
            pltpu.VMEM((NWIN, KW), jnp.int32),
            pltpu.VMEM((NWIN, KW), jnp.int32),
            pltpu.VMEM((MPT // MKW, MKW), jnp.int32),
            pltpu.VMEM((MPT // MKW, MKW), jnp.float32),
            pltpu.VMEM((KW,), jnp.float32),
            pltpu.VMEM((HPT,), jnp.float32),
            pltpu.VMEM((NH,), jnp.float32),
            pltpu.VMEM_SHARED((NH,), jnp.float32),
            pltpu.VMEM_SHARED((NH,), jnp.float32),
            pltpu.VMEM_SHARED((NH,), jnp.float32),
        ),
    )


def _make_prop(nchunks, ngather):
    def body(*refs):
        ys = refs[:nchunks]
        src3, dst3, z_h = refs[nchunks:nchunks + 3]
        k = nchunks + 3
        gidx_h = None
        gts = ()
        if ngather:
            gidx_h = refs[k]
            gts = refs[k + 1:k + 1 + ngather]
            k += 1 + ngather
        outs = refs[k:k + nchunks]
        k += nchunks
        gouts = refs[k:k + ngather]
        k += ngather
        sidx_v, didx_v, wbuf0_v, wbuf1_v, sem0, sem1 = refs[k:k + 6]
        if ngather:
            gidx_v = refs[k + 6]
        acc_sh = refs[-1]

        core = lax.axis_index("c")
        sid = lax.axis_index("s")
        wid = _wid()
        b0 = wbuf0_v
        b1 = wbuf1_v
        for c in range(nchunks):
            pltpu.sync_copy(z_h, wbuf0_v)
            for r in range(APT // FL):
                pltpu.sync_copy(
                    wbuf0_v, acc_sh.at[pl.ds(sid * APT + r * FL, FL)])
            plsc.subcore_barrier()
            for half in range(NWIN // HW):
                pltpu.sync_copy(src3.at[wid].at[pl.ds(half * HW, HW)], sidx_v)
                pltpu.sync_copy(dst3.at[wid].at[pl.ds(half * HW, HW)], didx_v)
                pltpu.async_copy(ys[c].at[sidx_v.at[0]], b0, sem0)

                def pair(i, carry):
                    j0 = 2 * i
                    pltpu.async_copy(ys[c].at[sidx_v.at[j0 + 1]], b1, sem1)
                    pltpu.make_async_copy(
                        ys[c].at[sidx_v.at[j0]], b0, sem0).wait()
                    pltpu.sync_copy(b0, acc_sh.at[didx_v.at[j0]], add=True)

                    @pl.when(i < HW // 2 - 1)
                    def _nx():
                        pltpu.async_copy(
                            ys[c].at[sidx_v.at[j0 + 2]], b0, sem0)

                    pltpu.make_async_copy(
                        ys[c].at[sidx_v.at[j0 + 1]], b1, sem1).wait()
                    pltpu.sync_copy(b1, acc_sh.at[didx_v.at[j0 + 1]], add=True)
                    return carry

                lax.fori_loop(0, HW // 2, pair, 0)
            plsc.subcore_barrier()
            for r in range(APT // FL):
                rows = pl.ds(sid * APT + r * FL, FL)
                pltpu.sync_copy(acc_sh.at[rows], wbuf0_v)
                pltpu.sync_copy(wbuf0_v, outs[c].at[core].at[rows])
            plsc.subcore_barrier()
        if ngather:
            pltpu.sync_copy(gidx_h.at[pl.ds(wid * MPT, MPT)], gidx_v)
            g0 = wbuf0_v.at[pl.ds(0, MKW)]
            g1 = wbuf1_v.at[pl.ds(0, MKW)]
            gsrcs = [gts[t].at[gidx_v.at[pl.ds(j * MKW, MKW)]]
                     for t in range(ngather) for j in range(MPT // MKW)]
            gdsts = [gouts[t].at[pl.ds(wid * MPT + j * MKW, MKW)]
                     for t in range(ngather) for j in range(MPT // MKW)]
            bufs = [g0, g1]
            sems = [sem0, sem1]
            pltpu.async_copy(gsrcs[0], bufs[0], sems[0])
            for i in range(len(gsrcs)):
                if i + 1 < len(gsrcs):
                    pltpu.async_copy(
                        gsrcs[i + 1], bufs[(i + 1) % 2], sems[(i + 1) % 2])
                pltpu.make_async_copy(gsrcs[i], bufs[i % 2], sems[i % 2]).wait()
                pltpu.sync_copy(bufs[i % 2], gdsts[i])

    out_type = tuple(
        jax.ShapeDtypeStruct((NC, NA, D), jnp.float32) for _ in range(nchunks)
    ) + tuple(
        jax.ShapeDtypeStruct((MP, D), jnp.float32) for _ in range(ngather)
    )
    scratch = [
        pltpu.VMEM((HW, KW), jnp.int32),
        pltpu.VMEM((HW, KW), jnp.int32),
        pltpu.VMEM((FL, D), jnp.float32),
        pltpu.VMEM((FL, D), jnp.float32),
        pltpu.SemaphoreType.DMA,
        pltpu.SemaphoreType.DMA,
    ]
    if ngather:
        scratch.append(pltpu.VMEM((MPT,), jnp.int32))
    scratch.append(pltpu.VMEM_SHARED((NA, D), jnp.float32))
    return pl.kernel(
        body,
        out_type=out_type,
        mesh=plsc.VectorSubcoreMesh(**_MESH),
        scratch_types=tuple(scratch),
    )


_make_prop = functools.cache(_make_prop)

NWIN2 = 2 * NWIN


def _prop2_body(y0, y1, srcT, dstT, z_h, out0, out1,
                sidx_v, didx_v, wbuf0_v, wbuf1_v, sem0, sem1, acc_sh):
    core = lax.axis_index("c")
    sid = lax.axis_index("s")
    pltpu.sync_copy(z_h, wbuf0_v)
    for r in range(APT // FL):
        pltpu.sync_copy(wbuf0_v, acc_sh.at[pl.ds(sid * APT + r * FL, FL)])
    plsc.subcore_barrier()
    for c in range(2):
        y = (y0, y1)[c]
        out = (out0, out1)[c]

        @pl.when(core == c)
        def _run(y=y, out=out):
            for ph in range(NWIN2 // HW):
                pltpu.sync_copy(srcT.at[sid].at[pl.ds(ph * HW, HW)], sidx_v)
                pltpu.sync_copy(dstT.at[sid].at[pl.ds(ph * HW, HW)], didx_v)
                pltpu.async_copy(y.at[sidx_v.at[0]], wbuf0_v, sem0)

                def pair(i, carry):
                    j0 = 2 * i
                    pltpu.async_copy(y.at[sidx_v.at[j0 + 1]], wbuf1_v, sem1)
                    pltpu.make_async_copy(
                        y.at[sidx_v.at[j0]], wbuf0_v, sem0).wait()
                    pltpu.sync_copy(
                        wbuf0_v, acc_sh.at[didx_v.at[j0]], add=True)

                    @pl.when(i < HW // 2 - 1)
                    def _nx():
                        pltpu.async_copy(
                            y.at[sidx_v.at[j0 + 2]], wbuf0_v, sem0)

                    pltpu.make_async_copy(
                        y.at[sidx_v.at[j0 + 1]], wbuf1_v, sem1).wait()
                    pltpu.sync_copy(
                        wbuf1_v, acc_sh.at[didx_v.at[j0 + 1]], add=True)
                    return carry

                lax.fori_loop(0, HW // 2, pair, 0)
            plsc.subcore_barrier()
            for r in range(APT // FL):
                rows = pl.ds(sid * APT + r * FL, FL)
                pltpu.sync_copy(acc_sh.at[rows], wbuf0_v)
                pltpu.sync_copy(wbuf0_v, out.at[rows])


@functools.cache
def _prop2_kernel():
    return pl.kernel(
        _prop2_body,
        out_type=(
            jax.ShapeDtypeStruct((NA, D), jnp.float32),
            jax.ShapeDtypeStruct((NA, D), jnp.float32),
        ),
        mesh=plsc.VectorSubcoreMesh(**_MESH),
        scratch_types=(
            pltpu.VMEM((HW, KW), jnp.int32),
            pltpu.VMEM((HW, KW), jnp.int32),
            pltpu.VMEM((FL, D), jnp.float32),
            pltpu.VMEM((FL, D), jnp.float32),
            pltpu.SemaphoreType.DMA,
            pltpu.SemaphoreType.DMA,
            pltpu.VMEM_SHARED((NA, D), jnp.float32),
        ),
    )


def _hist_call(*args):
    return _hist_kernel()(*args)


def _prop2(y0, y1, src3, dst3, z_h):
    srcT = src3.reshape(NS, NWIN2, KW)
    dstT = dst3.reshape(NS, NWIN2, KW)
    return _prop2_kernel()(y0, y1, srcT, dstT, z_h)


def _prop1g2(*args):
    return _make_prop(1, 2)(*args)


BR = 2000
NB = N // BR
BM = 512
NBM = MP // BM

_f32 = jnp.float32


def _row(shape):
    return pl.BlockSpec(shape, lambda p, i: (i, 0))


def _full2(shape):
    return pl.BlockSpec(shape, lambda p, i: tuple(0 for _ in shape))


def _pp(shape):
    return pl.BlockSpec(shape, lambda p, i: (0, i, 0))


def _prelu_(x, a_ref):
    return jnp.where(x >= 0, x, a_ref[0, 0] * x)


def _k1_body(sa, sb, dja, djb, ma, mb, feat_b, mt,
             ns_o, nd_o, m01_o, y0_o, y1_o):
    ns = jnp.clip(sa[...] + sb[...], 1.0, None) ** -0.5
    nd = jnp.clip(dja[...] + djb[...], 1.0, None) ** -0.5
    m01 = ma[...] + mb[...]
    f = feat_b[...]
    x = f * (1.0 - m01) + m01 * mt[...]
    ns_o[...] = ns
    nd_o[...] = nd
    m01_o[...] = m01
    y0_o[...] = x * ns
    y1_o[...] = f * ns


def _k1_call(*arrs):
    return pl.pallas_call(
        _k1_body,
        grid=(1, NB),
        in_specs=[_row((BR, 1))] * 6 + [_row((BR, D)), _full2((1, D))],
        out_specs=[_row((BR, 1))] * 3 + [_row((BR, D))] * 2,
        out_shape=[jax.ShapeDtypeStruct((N, 1), _f32)] * 3
        + [jax.ShapeDtypeStruct((NA, D), _f32)] * 2,
    )(*arrs)


def _bn_stats_acc(st, pre, r0):
    st[r0:r0 + 1, :] += jnp.sum(pre, 0, keepdims=True)
    st[r0 + 1:r0 + 2, :] += jnp.sum(pre * pre, 0, keepdims=True)


def _bn_apply(st, pre, r0, g, be, a):
    m = st[r0:r0 + 1, :] / N
    v = st[r0 + 1:r0 + 2, :] / N - m * m
    h = (pre - m) * lax.rsqrt(v + 1e-5) * g[...] + be[...]
    return _prelu_(h, a)


def _k2_body(p0, p1, ns, nd, W1, b1, g1, be1, a1, W2,
             tW1, tb1, tg1, tbe1, ta1, tW2, y2a_o, y2b_o, st):
    p = pl.program_id(0)
    i = pl.program_id(1)

    @pl.when((p == 0) & (i == 0))
    def _z():
        st[...] = jnp.zeros_like(st)

    ndb = nd[...]
    prex = (ndb * p0[...]) @ W1[...] + b1[...]
    pref = (ndb * p1[...]) @ tW1[...] + tb1[...]

    @pl.when(p == 0)
    def _acc():
        _bn_stats_acc(st, prex, 0)
        _bn_stats_acc(st, pref, 2)

    @pl.when(p == 1)
    def _apply():
        nsb = ns[...]
        e1 = _bn_apply(st, prex, 0, g1, be1, a1)
        te1 = _bn_apply(st, pref, 2, tg1, tbe1, ta1)
        y2a_o[...] = (e1 * nsb) @ W2[...]
        y2b_o[...] = (te1 * nsb) @ tW2[...]


def _k2_call(p0, p1, ns, nd, *ws):
    return pl.pallas_call(
        _k2_body,
        grid=(2, NB),
        in_specs=[_row((BR, D))] * 2 + [_row((BR, 1))] * 2
        + [_full2((D, H)), _full2((1, H)), _full2((1, H)), _full2((1, H)),
           _full2((1, 1)), _full2((H, D))] * 2,
        out_specs=[_row((BR, D))] * 2,
        out_shape=[jax.ShapeDtypeStruct((NA, D), _f32)] * 2,
        scratch_shapes=[pltpu.VMEM((8, H), _f32)],
    )(p0, p1, ns, nd, *ws)


def _k3_body(q0, q1, ns, nd, m01, b2, g2, be2, a2, tb2, tg2, tbe2, ta2, dW,
             o_o, h2_o, y3_o, st):
    p = pl.program_id(0)
    i = pl.program_id(1)

    @pl.when((p == 0) & (i == 0))
    def _z():
        st[...] = jnp.zeros_like(st)

    ndb = nd[...]
    preo = ndb * q0[...] + b2[...]
    preh = ndb * q1[...] + tb2[...]

    @pl.when(p == 0)
    def _acc():
        _bn_stats_acc(st, preo, 0)
        _bn_stats_acc(st, preh, 2)

    @pl.when(p == 1)
    def _apply():
        o = _bn_apply(st, preo, 0, g2, be2, a2)
        h2 = _bn_apply(st, preh, 2, tg2, tbe2, ta2)
        o_o[...] = o
        h2_o[...] = h2
        y3_o[...] = ((o * (1.0 - m01[...])) * ns[...]) @ dW[...]


def _k3_call(q0, q1, ns, nd, m01, *ws):
    return pl.pallas_call(
        _k3_body,
        grid=(2, NB),
        in_specs=[_row((BR, D))] * 2 + [_row((BR, 1))] * 3
        + [_full2((1, D))] * 3 + [_full2((1, 1))]
        + [_full2((1, D))] * 3 + [_full2((1, 1))] + [_full2((D, D))],
        out_specs=[_row((BR, D))] * 3,
        out_shape=[jax.ShapeDtypeStruct((NA, D), _f32)] * 3,
        scratch_shapes=[pltpu.VMEM((8, D), _f32)],
    )(q0, q1, ns, nd, m01, *ws)


def _k4_body(r0, nd, m01, feat_b, db, dg, dbe, da, loss_o, st, acc):
    p = pl.program_id(0)
    i = pl.program_id(1)

    @pl.when((p == 0) & (i == 0))
    def _z():
        st[...] = jnp.zeros_like(st)
        acc[0, 0] = 0.0

    u = nd[...] * (r0[0] + r0[1]) + db[...]

    @pl.when(p == 0)
    def _acc():
        _bn_stats_acc(st, u, 0)

    @pl.when(p == 1)
    def _apply():
        re = _bn_apply(st, u, 0, dg, dbe, da)
        fb = feat_b[...]
        rn = jnp.maximum(jnp.sqrt(jnp.sum(re * re, 1, keepdims=True)), 1e-12)
        fn = jnp.maximum(jnp.sqrt(jnp.sum(fb * fb, 1, keepdims=True)), 1e-12)
        cos = jnp.sum(re * fb, 1, keepdims=True) / (rn * fn)
        acc[0, 0] += jnp.sum(m01[...] * (1.0 - cos))

    @pl.when((p == 1) & (i == NB - 1))
    def _fin():
        loss_o[...] = jnp.full((1, 1), acc[0, 0] / MN, _f32)


def _k4_call(r0, nd, m01, feat, db, dg, dbe, da):
    return pl.pallas_call(
        _k4_body,
        grid=(2, NB),
        in_specs=[_pp((NC, BR, D))] + [_row((BR, 1))] * 2 + [_row((BR, D))]
        + [_full2((1, D))] * 3 + [_full2((1, 1))],
        out_specs=pl.BlockSpec((1, 1), lambda p, i: (0, 0)),
        out_shape=jax.ShapeDtypeStruct((1, 1), _f32),
        scratch_shapes=[pltpu.VMEM((8, D), _f32),
                        pltpu.SMEM((1, 1), _f32)],
    )(r0, nd, m01, feat, db, dg, dbe, da)


def _head(xb, W1_, b1_, W2_, b2_):
    t = jnp.maximum(xb @ W1_[...] + b1_[...], 0.0)
    c = t @ W2_[...] + b2_[...]
    n = jnp.maximum(jnp.sqrt(jnp.sum(c * c, 1, keepdims=True)), 1e-12)
    return c / n


def _k5_body(hm, om, pW1, pb1, pW2, pb2, qW1, qb1, qW2, qb2, nh_o, nm_o):
    nh_o[...] = _head(hm[...], pW1, pb1, pW2, pb2)
    nm_o[...] = _head(om[...], qW1, qb1, qW2, qb2)


def _k5_call(hm, om, *ws):
    return pl.pallas_call(
        _k5_body,
        grid=(1, NBM),
        in_specs=[_row((BM, D))] * 2
        + [_full2((D, H)), _full2((1, H)), _full2((H, D)), _full2((1, D))] * 2,
        out_specs=[_row((BM, D))] * 2,
        out_shape=[jax.ShapeDtypeStruct((MP, D), _f32)] * 2,
    )(hm, om, *ws)


def _k6_body(nh_b, nm_full, nm_b, loss1, out_o, acc):
    i = pl.program_id(1)

    @pl.when(i == 0)
    def _z():
        acc[0, 0] = 0.0

    a = nh_b[...]
    s = lax.dot_general(a, nm_full[...], (((1,), (1,)), ((), ())),
                        preferred_element_type=_f32) / T
    sim = jnp.exp(s)
    colm = (lax.broadcasted_iota(jnp.int32, (BM, MP), 1) < MN).astype(_f32)
    rowsum = jnp.sum(sim * colm, 1, keepdims=True)
    pos = jnp.exp(jnp.sum(a * nm_b[...], 1, keepdims=True) / T)
    rowid = i * BM + lax.broadcasted_iota(jnp.int32, (BM, 1), 0)
    valid = (rowid < MN).astype(_f32)
    den = jnp.where(valid > 0, rowsum - pos, 1.0)
    acc[0, 0] += jnp.sum(valid * (jnp.log(den) - jnp.log(pos)))

    @pl.when(i == NBM - 1)
    def _fin():
        out_o[...] = loss1[...] + ALPHA * jnp.full((1, 1), acc[0, 0] / MN, _f32)


def _k6_call(nh, nm, loss1):
    return pl.pallas_call(
        _k6_body,
        grid=(1, NBM),
        in_specs=[_row((BM, D)), _full2((MP, D)), _row((BM, D)),
                  _full2((1, 1))],
        out_specs=pl.BlockSpec((1, 1), lambda p, i: (0, 0)),
        out_shape=jax.ShapeDtypeStruct((1, 1), _f32),
        scratch_shapes=[pltpu.SMEM((1, 1), _f32)],
    )(nh, nm, nm, loss1)


def kernel(feat, edge_index, mask_nodes, W1, b1, g1, be1, a1, W2, b2, g2, be2,
           a2, tW1, tb1, tg1, tbe1, ta1, tW2, tb2, tg2, tbe2, ta2,
           dW, db, dg, dbe, da, mask_token,
           pW1, pb1, pW2, pb2, qW1, qb1, qW2, qb2):
    dummy = (N + jnp.arange(EPP - EPT, dtype=jnp.int32))[None, :]
    dummy = jnp.broadcast_to(dummy, (NW, EPP - EPT))
    src3 = jnp.concatenate(
        [edge_index[0].reshape(NW, EPT), dummy], axis=1).reshape(NW, NWIN, KW)
    dst3 = jnp.concatenate(
        [edge_index[1].reshape(NW, EPT), dummy], axis=1).reshape(NW, NWIN, KW)
    mpad = jnp.concatenate(
        [mask_nodes, jnp.zeros((MP - MN,), jnp.int32)])
    msk3 = mpad.reshape(NW, MPT // MKW, MKW)
    mupd3 = jnp.concatenate(
        [jnp.ones((MN,), jnp.float32), jnp.zeros((MP - MN,), jnp.float32)]
    ).reshape(NW, MPT // MKW, MKW)
    ones_h = jnp.ones((KW,), jnp.float32)
    zh_h = jnp.zeros((HPT,), jnp.float32)
    zr_h = jnp.zeros((FL, D), jnp.float32)

    degs2, degd2, m012 = _hist_call(src3, dst3, msk3, mupd3, ones_h, zh_h)
    col = lambda a: a.reshape(NH, 1)[:N]
    row = lambda a: a.reshape(1, -1)
    sc = lambda a: a.reshape(1, 1)

    ns, nd, m01, y0, y1 = _k1_call(
        col(degs2[0]), col(degs2[1]), col(degd2[0]), col(degd2[1]),
        col(m012[0]), col(m012[1]), feat, mask_token)

    p0, p1 = _prop2(y0, y1, src3, dst3, zr_h)
    y2a, y2b = _k2_call(
        p0, p1, ns, nd,
        W1, row(b1), row(g1), row(be1), sc(a1), W2,
        tW1, row(tb1), row(tg1), row(tbe1), sc(ta1), tW2)

    q0, q1 = _prop2(y2a, y2b, src3, dst3, zr_h)
    o, h2, y3 = _k3_call(
        q0, q1, ns, nd, m01,
        row(b2), row(g2), row(be2), sc(a2),
        row(tb2), row(tg2), row(tbe2), sc(ta2), dW)

    r0, om, hm = _prop1g2(y3, src3, dst3, zr_h, mpad, o, h2)
    loss1 = _k4_call(r0, nd, m01, feat, row(db), row(dg), row(dbe), sc(da))

    nh, nm = _k5_call(hm, om, pW1, row(pb1), pW2, row(pb2),
                      qW1, row(qb1), qW2, row(qb2))
    total = _k6_call(nh, nm, loss1)
    return total[0, 0]


# speedup vs baseline: 1.0683x; 1.0121x over previous
import functools

import jax
import jax.numpy as jnp
from jax import lax
from jax.experimental import pallas as pl
from jax.experimental.pallas import tpu as pltpu
from jax.experimental.pallas import tpu_sc as plsc

N = 10000
E = 320000
D = 128
H = 256
T = 0.2
ALPHA = 0.5

NC, NS = 2, 16
NW = NC * NS
EPT = E // NW
KW = 128
NWIN = 80
EPP = NWIN * KW
HW = NWIN // 2
MN = 5000
MP = 5120
MPT = MP // NW
MKW = 80
NH = 10240
HPT = NH // NS
NA = 10240
APT = NA // NS
FL = 128

_MESH = dict(core_axis_name="c", subcore_axis_name="s")


def _wid():
    return lax.axis_index("s") * NC + lax.axis_index("c")


def _hist_body(src3, dst3, msk3, mupd3, ones_h, z_h,
               degs_o, degd_o, m01_o,
               sidx_v, didx_v, midx_v, mupd_v, ones_v, z_v, bounce_v, sem,
               hs_sh, hd_sh, hm_sh):
    core = lax.axis_index("c")
    sid = lax.axis_index("s")
    wid = _wid()
    pltpu.sync_copy(z_h, z_v)
    pltpu.sync_copy(z_v, hs_sh.at[pl.ds(sid * HPT, HPT)])
    pltpu.sync_copy(z_v, hd_sh.at[pl.ds(sid * HPT, HPT)])
    pltpu.sync_copy(z_v, hm_sh.at[pl.ds(sid * HPT, HPT)])
    pltpu.sync_copy(ones_h, ones_v)
    pltpu.sync_copy(src3.at[wid], sidx_v)
    pltpu.sync_copy(dst3.at[wid], didx_v)
    pltpu.sync_copy(msk3.at[wid], midx_v)
    pltpu.sync_copy(mupd3.at[wid], mupd_v)
    plsc.subcore_barrier()

    def win(j, carry):
        pltpu.async_copy(ones_v, hs_sh.at[sidx_v.at[j]], sem, add=True)
        pltpu.async_copy(ones_v, hd_sh.at[didx_v.at[j]], sem, add=True)
        return carry

    lax.fori_loop(0, NWIN, win, 0)
    pltpu.async_copy(mupd_v.at[0], hm_sh.at[midx_v.at[0]], sem, add=True)
    pltpu.async_copy(mupd_v.at[1], hm_sh.at[midx_v.at[1]], sem, add=True)

    def drain(j, carry):
        pltpu.make_async_copy(ones_v, hs_sh.at[sidx_v.at[0]], sem).wait()
        pltpu.make_async_copy(ones_v, hd_sh.at[sidx_v.at[0]], sem).wait()
        return carry

    lax.fori_loop(0, NWIN, drain, 0)
    pltpu.make_async_copy(mupd_v.at[0], hm_sh.at[midx_v.at[0]], sem).wait()
    pltpu.make_async_copy(mupd_v.at[1], hm_sh.at[midx_v.at[1]], sem).wait()
    plsc.subcore_barrier()

    @pl.when(sid == 0)
    def _f0():
        pltpu.sync_copy(hs_sh, bounce_v)
        pltpu.sync_copy(bounce_v, degs_o.at[core])

    @pl.when(sid == 1)
    def _f1():
        pltpu.sync_copy(hd_sh, bounce_v)
        pltpu.sync_copy(bounce_v, degd_o.at[core])

    @pl.when(sid == 2)
    def _f2():
        pltpu.sync_copy(hm_sh, bounce_v)
        pltpu.sync_copy(bounce_v, m01_o.at[core])


@functools.cache
def _hist_kernel():
    return pl.kernel(
        _hist_body,
        out_type=(
            jax.ShapeDtypeStruct((NC, NH), jnp.float32),
            jax.ShapeDtypeStruct((NC, NH), jnp.float32),
            jax.ShapeDtypeStruct((NC, NH), jnp.float32),
        ),
        mesh=plsc.VectorSubcoreMesh(**_MESH),
        scratch_types=(
            pltpu.VMEM((NWIN, KW), jnp.int32),
            pltpu.VMEM((NWIN, KW), jnp.int32),
            pltpu.VMEM((MPT // MKW, MKW), jnp.int32),
            pltpu.VMEM((MPT // MKW, MKW), jnp.float32),
            pltpu.VMEM((KW,), jnp.float32),
            pltpu.VMEM((HPT,), jnp.float32),
            pltpu.VMEM((NH,), jnp.float32),
            pltpu.SemaphoreType.DMA,
            pltpu.VMEM_SHARED((NH,), jnp.float32),
            pltpu.VMEM_SHARED((NH,), jnp.float32),
            pltpu.VMEM_SHARED((NH,), jnp.float32),
        ),
    )


def _make_prop(nchunks, ngather):
    def body(*refs):
        ys = refs[:nchunks]
        src3, dst3, z_h = refs[nchunks:nchunks + 3]
        k = nchunks + 3
        gidx_h = None
        gts = ()
        if ngather:
            gidx_h = refs[k]
            gts = refs[k + 1:k + 1 + ngather]
            k += 1 + ngather
        outs = refs[k:k + nchunks]
        k += nchunks
        gouts = refs[k:k + ngather]
        k += ngather
        sidx_v, didx_v, wbuf0_v, wbuf1_v, sem0, sem1 = refs[k:k + 6]
        if ngather:
            gidx_v = refs[k + 6]
        acc_sh = refs[-1]

        core = lax.axis_index("c")
        sid = lax.axis_index("s")
        wid = _wid()
        b0 = wbuf0_v
        b1 = wbuf1_v
        for c in range(nchunks):
            pltpu.sync_copy(z_h, wbuf0_v)
            for r in range(APT // FL):
                pltpu.sync_copy(
                    wbuf0_v, acc_sh.at[pl.ds(sid * APT + r * FL, FL)])
            plsc.subcore_barrier()
            for half in range(NWIN // HW):
                pltpu.sync_copy(src3.at[wid].at[pl.ds(half * HW, HW)], sidx_v)
                pltpu.sync_copy(dst3.at[wid].at[pl.ds(half * HW, HW)], didx_v)
                pltpu.async_copy(ys[c].at[sidx_v.at[0]], b0, sem0)

                def pair(i, carry):
                    j0 = 2 * i
                    pltpu.async_copy(ys[c].at[sidx_v.at[j0 + 1]], b1, sem1)
                    pltpu.make_async_copy(
                        ys[c].at[sidx_v.at[j0]], b0, sem0).wait()
                    pltpu.sync_copy(b0, acc_sh.at[didx_v.at[j0]], add=True)

                    @pl.when(i < HW // 2 - 1)
                    def _nx():
                        pltpu.async_copy(
                            ys[c].at[sidx_v.at[j0 + 2]], b0, sem0)

                    pltpu.make_async_copy(
                        ys[c].at[sidx_v.at[j0 + 1]], b1, sem1).wait()
                    pltpu.sync_copy(b1, acc_sh.at[didx_v.at[j0 + 1]], add=True)
                    return carry

                lax.fori_loop(0, HW // 2, pair, 0)
            plsc.subcore_barrier()
            for r in range(APT // FL):
                rows = pl.ds(sid * APT + r * FL, FL)
                pltpu.sync_copy(acc_sh.at[rows], wbuf0_v)
                pltpu.sync_copy(wbuf0_v, outs[c].at[core].at[rows])
            plsc.subcore_barrier()
        if ngather:
            pltpu.sync_copy(gidx_h.at[pl.ds(wid * MPT, MPT)], gidx_v)
            g0 = wbuf0_v.at[pl.ds(0, MKW)]
            g1 = wbuf1_v.at[pl.ds(0, MKW)]
            gsrcs = [gts[t].at[gidx_v.at[pl.ds(j * MKW, MKW)]]
                     for t in range(ngather) for j in range(MPT // MKW)]
            gdsts = [gouts[t].at[pl.ds(wid * MPT + j * MKW, MKW)]
                     for t in range(ngather) for j in range(MPT // MKW)]
            bufs = [g0, g1]
            sems = [sem0, sem1]
            pltpu.async_copy(gsrcs[0], bufs[0], sems[0])
            for i in range(len(gsrcs)):
                if i + 1 < len(gsrcs):
                    pltpu.async_copy(
                        gsrcs[i + 1], bufs[(i + 1) % 2], sems[(i + 1) % 2])
                pltpu.make_async_copy(gsrcs[i], bufs[i % 2], sems[i % 2]).wait()
                pltpu.sync_copy(bufs[i % 2], gdsts[i])

    out_type = tuple(
        jax.ShapeDtypeStruct((NC, NA, D), jnp.float32) for _ in range(nchunks)
    ) + tuple(
        jax.ShapeDtypeStruct((MP, D), jnp.float32) for _ in range(ngather)
    )
    scratch = [
        pltpu.VMEM((HW, KW), jnp.int32),
        pltpu.VMEM((HW, KW), jnp.int32),
        pltpu.VMEM((FL, D), jnp.float32),
        pltpu.VMEM((FL, D), jnp.float32),
        pltpu.SemaphoreType.DMA,
        pltpu.SemaphoreType.DMA,
    ]
    if ngather:
        scratch.append(pltpu.VMEM((MPT,), jnp.int32))
    scratch.append(pltpu.VMEM_SHARED((NA, D), jnp.float32))
    return pl.kernel(
        body,
        out_type=out_type,
        mesh=plsc.VectorSubcoreMesh(**_MESH),
        scratch_types=tuple(scratch),
    )


_make_prop = functools.cache(_make_prop)

NWIN2 = 2 * NWIN


def _prop2_body(y0, y1, srcT, dstT, z_h, out0, out1,
                sidx_v, didx_v, wbuf0_v, wbuf1_v, sem0, sem1, acc_sh):
    core = lax.axis_index("c")
    sid = lax.axis_index("s")
    pltpu.sync_copy(z_h, wbuf0_v)
    for r in range(APT // FL):
        pltpu.sync_copy(wbuf0_v, acc_sh.at[pl.ds(sid * APT + r * FL, FL)])
    plsc.subcore_barrier()
    for c in range(2):
        y = (y0, y1)[c]
        out = (out0, out1)[c]

        @pl.when(core == c)
        def _run(y=y, out=out):
            for ph in range(NWIN2 // HW):
                pltpu.sync_copy(srcT.at[sid].at[pl.ds(ph * HW, HW)], sidx_v)
                pltpu.sync_copy(dstT.at[sid].at[pl.ds(ph * HW, HW)], didx_v)
                pltpu.async_copy(y.at[sidx_v.at[0]], wbuf0_v, sem0)

                def pair(i, carry):
                    j0 = 2 * i
                    pltpu.async_copy(y.at[sidx_v.at[j0 + 1]], wbuf1_v, sem1)
                    pltpu.make_async_copy(
                        y.at[sidx_v.at[j0]], wbuf0_v, sem0).wait()
                    pltpu.sync_copy(
                        wbuf0_v, acc_sh.at[didx_v.at[j0]], add=True)

                    @pl.when(i < HW // 2 - 1)
                    def _nx():
                        pltpu.async_copy(
                            y.at[sidx_v.at[j0 + 2]], wbuf0_v, sem0)

                    pltpu.make_async_copy(
                        y.at[sidx_v.at[j0 + 1]], wbuf1_v, sem1).wait()
                    pltpu.sync_copy(
                        wbuf1_v, acc_sh.at[didx_v.at[j0 + 1]], add=True)
                    return carry

                lax.fori_loop(0, HW // 2, pair, 0)
            plsc.subcore_barrier()
            for r in range(APT // FL):
                rows = pl.ds(sid * APT + r * FL, FL)
                pltpu.sync_copy(acc_sh.at[rows], wbuf0_v)
                pltpu.sync_copy(wbuf0_v, out.at[rows])


@functools.cache
def _prop2_kernel():
    return pl.kernel(
        _prop2_body,
        out_type=(
            jax.ShapeDtypeStruct((NA, D), jnp.float32),
            jax.ShapeDtypeStruct((NA, D), jnp.float32),
        ),
        mesh=plsc.VectorSubcoreMesh(**_MESH),
        scratch_types=(
            pltpu.VMEM((HW, KW), jnp.int32),
            pltpu.VMEM((HW, KW), jnp.int32),
            pltpu.VMEM((FL, D), jnp.float32),
            pltpu.VMEM((FL, D), jnp.float32),
            pltpu.SemaphoreType.DMA,
            pltpu.SemaphoreType.DMA,
            pltpu.VMEM_SHARED((NA, D), jnp.float32),
        ),
    )


def _hist_call(*args):
    return _hist_kernel()(*args)


def _prop2(y0, y1, src3, dst3, z_h):
    srcT = src3.reshape(NS, NWIN2, KW)
    dstT = dst3.reshape(NS, NWIN2, KW)
    return _prop2_kernel()(y0, y1, srcT, dstT, z_h)


def _prop1g2(*args):
    return _make_prop(1, 2)(*args)


BR = 2000
NB = N // BR
BM = 512
NBM = MP // BM

_f32 = jnp.float32


def _row(shape):
    return pl.BlockSpec(shape, lambda p, i: (i, 0))


def _full2(shape):
    return pl.BlockSpec(shape, lambda p, i: tuple(0 for _ in shape))


def _pp(shape):
    return pl.BlockSpec(shape, lambda p, i: (0, i, 0))


def _prelu_(x, a_ref):
    return jnp.where(x >= 0, x, a_ref[0, 0] * x)


def _k1_body(sa, sb, dja, djb, ma, mb, feat_b, mt,
             ns_o, nd_o, m01_o, y0_o, y1_o):
    ns = jnp.clip(sa[...] + sb[...], 1.0, None) ** -0.5
    nd = jnp.clip(dja[...] + djb[...], 1.0, None) ** -0.5
    m01 = ma[...] + mb[...]
    f = feat_b[...]
    x = f * (1.0 - m01) + m01 * mt[...]
    ns_o[...] = ns
    nd_o[...] = nd
    m01_o[...] = m01
    y0_o[...] = x * ns
    y1_o[...] = f * ns


def _k1_call(*arrs):
    return pl.pallas_call(
        _k1_body,
        grid=(1, NB),
        in_specs=[_row((BR, 1))] * 6 + [_row((BR, D)), _full2((1, D))],
        out_specs=[_row((BR, 1))] * 3 + [_row((BR, D))] * 2,
        out_shape=[jax.ShapeDtypeStruct((N, 1), _f32)] * 3
        + [jax.ShapeDtypeStruct((NA, D), _f32)] * 2,
    )(*arrs)


def _bn_stats_acc(st, pre, r0):
    st[r0:r0 + 1, :] += jnp.sum(pre, 0, keepdims=True)
    st[r0 + 1:r0 + 2, :] += jnp.sum(pre * pre, 0, keepdims=True)


def _bn_apply(st, pre, r0, g, be, a):
    m = st[r0:r0 + 1, :] / N
    v = st[r0 + 1:r0 + 2, :] / N - m * m
    h = (pre - m) * lax.rsqrt(v + 1e-5) * g[...] + be[...]
    return _prelu_(h, a)


def _k2_body(p0, p1, ns, nd, W1, b1, g1, be1, a1, W2,
             tW1, tb1, tg1, tbe1, ta1, tW2, y2a_o, y2b_o, st):
    p = pl.program_id(0)
    i = pl.program_id(1)

    @pl.when((p == 0) & (i == 0))
    def _z():
        st[...] = jnp.zeros_like(st)

    ndb = nd[...]
    prex = (ndb * p0[...]) @ W1[...] + b1[...]
    pref = (ndb * p1[...]) @ tW1[...] + tb1[...]

    @pl.when(p == 0)
    def _acc():
        _bn_stats_acc(st, prex, 0)
        _bn_stats_acc(st, pref, 2)

    @pl.when(p == 1)
    def _apply():
        nsb = ns[...]
        e1 = _bn_apply(st, prex, 0, g1, be1, a1)
        te1 = _bn_apply(st, pref, 2, tg1, tbe1, ta1)
        y2a_o[...] = (e1 * nsb) @ W2[...]
        y2b_o[...] = (te1 * nsb) @ tW2[...]


def _k2_call(p0, p1, ns, nd, *ws):
    return pl.pallas_call(
        _k2_body,
        grid=(2, NB),
        in_specs=[_row((BR, D))] * 2 + [_row((BR, 1))] * 2
        + [_full2((D, H)), _full2((1, H)), _full2((1, H)), _full2((1, H)),
           _full2((1, 1)), _full2((H, D))] * 2,
        out_specs=[_row((BR, D))] * 2,
        out_shape=[jax.ShapeDtypeStruct((NA, D), _f32)] * 2,
        scratch_shapes=[pltpu.VMEM((8, H), _f32)],
    )(p0, p1, ns, nd, *ws)


def _k3_body(q0, q1, ns, nd, m01, b2, g2, be2, a2, tb2, tg2, tbe2, ta2, dW,
             o_o, h2_o, y3_o, st):
    p = pl.program_id(0)
    i = pl.program_id(1)

    @pl.when((p == 0) & (i == 0))
    def _z():
        st[...] = jnp.zeros_like(st)

    ndb = nd[...]
    preo = ndb * q0[...] + b2[...]
    preh = ndb * q1[...] + tb2[...]

    @pl.when(p == 0)
    def _acc():
        _bn_stats_acc(st, preo, 0)
        _bn_stats_acc(st, preh, 2)

    @pl.when(p == 1)
    def _apply():
        o = _bn_apply(st, preo, 0, g2, be2, a2)
        h2 = _bn_apply(st, preh, 2, tg2, tbe2, ta2)
        o_o[...] = o
        h2_o[...] = h2
        y3_o[...] = ((o * (1.0 - m01[...])) * ns[...]) @ dW[...]


def _k3_call(q0, q1, ns, nd, m01, *ws):
    return pl.pallas_call(
        _k3_body,
        grid=(2, NB),
        in_specs=[_row((BR, D))] * 2 + [_row((BR, 1))] * 3
        + [_full2((1, D))] * 3 + [_full2((1, 1))]
        + [_full2((1, D))] * 3 + [_full2((1, 1))] + [_full2((D, D))],
        out_specs=[_row((BR, D))] * 3,
        out_shape=[jax.ShapeDtypeStruct((NA, D), _f32)] * 3,
        scratch_shapes=[pltpu.VMEM((8, D), _f32)],
    )(q0, q1, ns, nd, m01, *ws)


def _k4_body(r0, nd, m01, feat_b, db, dg, dbe, da, loss_o, st, acc):
    p = pl.program_id(0)
    i = pl.program_id(1)

    @pl.when((p == 0) & (i == 0))
    def _z():
        st[...] = jnp.zeros_like(st)
        acc[0, 0] = 0.0

    u = nd[...] * (r0[0] + r0[1]) + db[...]

    @pl.when(p == 0)
    def _acc():
        _bn_stats_acc(st, u, 0)

    @pl.when(p == 1)
    def _apply():
        re = _bn_apply(st, u, 0, dg, dbe, da)
        fb = feat_b[...]
        rn = jnp.maximum(jnp.sqrt(jnp.sum(re * re, 1, keepdims=True)), 1e-12)
        fn = jnp.maximum(jnp.sqrt(jnp.sum(fb * fb, 1, keepdims=True)), 1e-12)
        cos = jnp.sum(re * fb, 1, keepdims=True) / (rn * fn)
        acc[0, 0] += jnp.sum(m01[...] * (1.0 - cos))

    @pl.when((p == 1) & (i == NB - 1))
    def _fin():
        loss_o[...] = jnp.full((1, 1), acc[0, 0] / MN, _f32)


def _k4_call(r0, nd, m01, feat, db, dg, dbe, da):
    return pl.pallas_call(
        _k4_body,
        grid=(2, NB),
        in_specs=[_pp((NC, BR, D))] + [_row((BR, 1))] * 2 + [_row((BR, D))]
        + [_full2((1, D))] * 3 + [_full2((1, 1))],
        out_specs=pl.BlockSpec((1, 1), lambda p, i: (0, 0)),
        out_shape=jax.ShapeDtypeStruct((1, 1), _f32),
        scratch_shapes=[pltpu.VMEM((8, D), _f32),
                        pltpu.SMEM((1, 1), _f32)],
    )(r0, nd, m01, feat, db, dg, dbe, da)


def _head(xb, W1_, b1_, W2_, b2_):
    t = jnp.maximum(xb @ W1_[...] + b1_[...], 0.0)
    c = t @ W2_[...] + b2_[...]
    n = jnp.maximum(jnp.sqrt(jnp.sum(c * c, 1, keepdims=True)), 1e-12)
    return c / n


def _k5_body(hm, om, pW1, pb1, pW2, pb2, qW1, qb1, qW2, qb2, nh_o, nm_o):
    nh_o[...] = _head(hm[...], pW1, pb1, pW2, pb2)
    nm_o[...] = _head(om[...], qW1, qb1, qW2, qb2)


def _k5_call(hm, om, *ws):
    return pl.pallas_call(
        _k5_body,
        grid=(1, NBM),
        in_specs=[_row((BM, D))] * 2
        + [_full2((D, H)), _full2((1, H)), _full2((H, D)), _full2((1, D))] * 2,
        out_specs=[_row((BM, D))] * 2,
        out_shape=[jax.ShapeDtypeStruct((MP, D), _f32)] * 2,
    )(hm, om, *ws)


def _k6_body(nh_b, nm_full, nm_b, loss1, out_o, acc):
    i = pl.program_id(1)

    @pl.when(i == 0)
    def _z():
        acc[0, 0] = 0.0

    a = nh_b[...]
    s = lax.dot_general(a, nm_full[...], (((1,), (1,)), ((), ())),
                        preferred_element_type=_f32) / T
    sim = jnp.exp(s)
    colm = (lax.broadcasted_iota(jnp.int32, (BM, MP), 1) < MN).astype(_f32)
    rowsum = jnp.sum(sim * colm, 1, keepdims=True)
    pos = jnp.exp(jnp.sum(a * nm_b[...], 1, keepdims=True) / T)
    rowid = i * BM + lax.broadcasted_iota(jnp.int32, (BM, 1), 0)
    valid = (rowid < MN).astype(_f32)
    den = jnp.where(valid > 0, rowsum - pos, 1.0)
    acc[0, 0] += jnp.sum(valid * (jnp.log(den) - jnp.log(pos)))

    @pl.when(i == NBM - 1)
    def _fin():
        out_o[...] = loss1[...] + ALPHA * jnp.full((1, 1), acc[0, 0] / MN, _f32)


def _k6_call(nh, nm, loss1):
    return pl.pallas_call(
        _k6_body,
        grid=(1, NBM),
        in_specs=[_row((BM, D)), _full2((MP, D)), _row((BM, D)),
                  _full2((1, 1))],
        out_specs=pl.BlockSpec((1, 1), lambda p, i: (0, 0)),
        out_shape=jax.ShapeDtypeStruct((1, 1), _f32),
        scratch_shapes=[pltpu.SMEM((1, 1), _f32)],
    )(nh, nm, nm, loss1)


def kernel(feat, edge_index, mask_nodes, W1, b1, g1, be1, a1, W2, b2, g2, be2,
           a2, tW1, tb1, tg1, tbe1, ta1, tW2, tb2, tg2, tbe2, ta2,
           dW, db, dg, dbe, da, mask_token,
           pW1, pb1, pW2, pb2, qW1, qb1, qW2, qb2):
    dummy = (N + jnp.arange(EPP - EPT, dtype=jnp.int32))[None, :]
    dummy = jnp.broadcast_to(dummy, (NW, EPP - EPT))
    src3 = jnp.concatenate(
        [edge_index[0].reshape(NW, EPT), dummy], axis=1).reshape(NW, NWIN, KW)
    dst3 = jnp.concatenate(
        [edge_index[1].reshape(NW, EPT), dummy], axis=1).reshape(NW, NWIN, KW)
    mpad = jnp.concatenate(
        [mask_nodes, jnp.zeros((MP - MN,), jnp.int32)])
    msk3 = mpad.reshape(NW, MPT // MKW, MKW)
    mupd3 = jnp.concatenate(
        [jnp.ones((MN,), jnp.float32), jnp.zeros((MP - MN,), jnp.float32)]
    ).reshape(NW, MPT // MKW, MKW)
    ones_h = jnp.ones((KW,), jnp.float32)
    zh_h = jnp.zeros((HPT,), jnp.float32)
    zr_h = jnp.zeros((FL, D), jnp.float32)

    degs2, degd2, m012 = _hist_call(src3, dst3, msk3, mupd3, ones_h, zh_h)
    col = lambda a: a.reshape(NH, 1)[:N]
    row = lambda a: a.reshape(1, -1)
    sc = lambda a: a.reshape(1, 1)

    ns, nd, m01, y0, y1 = _k1_call(
        col(degs2[0]), col(degs2[1]), col(degd2[0]), col(degd2[1]),
        col(m012[0]), col(m012[1]), feat, mask_token)

    p0, p1 = _prop2(y0, y1, src3, dst3, zr_h)
    y2a, y2b = _k2_call(
        p0, p1, ns, nd,
        W1, row(b1), row(g1), row(be1), sc(a1), W2,
        tW1, row(tb1), row(tg1), row(tbe1), sc(ta1), tW2)

    q0, q1 = _prop2(y2a, y2b, src3, dst3, zr_h)
    o, h2, y3 = _k3_call(
        q0, q1, ns, nd, m01,
        row(b2), row(g2), row(be2), sc(a2),
        row(tb2), row(tg2), row(tbe2), sc(ta2), dW)

    r0, om, hm = _prop1g2(y3, src3, dst3, zr_h, mpad, o, h2)
    loss1 = _k4_call(r0, nd, m01, feat, row(db), row(dg), row(dbe), sc(da))

    nh, nm = _k5_call(hm, om, pW1, row(pb1), pW2, row(pb2),
                      qW1, row(qb1), qW2, row(qb2))
    total = _k6_call(nh, nm, loss1)
    return total[0, 0]
